# baseline (speedup 1.0000x reference)
import dataclasses
import functools

import jax
import jax.numpy as jnp
from jax import lax
from jax.experimental import pallas as pl
from jax.experimental.pallas import tpu as pltpu
from jax.experimental.pallas import tpu_sc as plsc

G = 30
H = 12
T = 2048
SPAN_C = 4 * G + 1
REL = (2 * G - 1) * SPAN_C

NUM_CORES = 2
NUM_SUBCORES = 16
NW = NUM_CORES * NUM_SUBCORES
ROWS_PER = T // NW
LANES = 16
CHUNKS = T // LANES
WPAD = 7


def _sc_body(row_hbm, col_hbm, qi_hbm, qj_hbm, tab_hbm, out_hbm,
             tab_v, row_v, col_v, qj_v, qi_v, buf0, buf1,
             sem_in, sem0, sem1):
  cid = lax.axis_index("c")
  sid = lax.axis_index("s")
  wid = sid * NUM_CORES + cid

  stage = [pltpu.make_async_copy(src, dst, sem_in)
           for src, dst in ((tab_hbm, tab_v), (row_hbm, row_v),
                            (col_hbm, col_v), (qj_hbm, qj_v), (qi_hbm, qi_v))]
  for cp_ in stage:
    cp_.start()
  for cp_ in stage:
    cp_.wait()

  def fill_and_send(r, buf, sem):
    i = r * NW + wid
    bidx = jnp.zeros((LANES,), jnp.int32) + i
    bri = plsc.load_gather(row_v, [bidx])
    bci = plsc.load_gather(col_v, [bidx])
    bqi = plsc.load_gather(qi_v, [bidx])

    @plsc.parallel_loop(0, CHUNKS, unroll=4)
    def _(c):
      j0 = c * LANES
      rj = row_v[pl.ds(j0, LANES)]
      cj = col_v[pl.ds(j0, LANES)]
      qj = qj_v[pl.ds(j0, LANES)]
      dr = jnp.clip(bri - rj, -(G - 1), G - 1) + (G - 1)
      dc = jnp.clip(bci - cj, -2 * G, 2 * G) + 2 * G
      base = jnp.where(bqi == qj, (dr * SPAN_C + dc) * WPAD, REL * WPAD)
      for p in range(H // 2):
        w = plsc.load_gather(tab_v, [base + p])
        buf[2 * p, pl.ds(c * LANES, LANES)] = plsc.bitcast(
            w << 16, jnp.float32)
        buf[2 * p + 1, pl.ds(c * LANES, LANES)] = plsc.bitcast(
            w & jnp.int32(-65536), jnp.float32)

    for h in range(H):
      pltpu.make_async_copy(buf.at[h], out_hbm.at[h, i, :], sem).start()

  def drain(buf, sem):
    for h in range(H):
      pltpu.make_async_copy(buf.at[h], out_hbm.at[h, 0, :], sem).wait()

  @pl.loop(0, ROWS_PER, step=2)
  def _(r):
    for d, buf, sem in ((0, buf0, sem0), (1, buf1, sem1)):
      @pl.when(r > 0)
      def _():
        drain(buf, sem)
      fill_and_send(r + d, buf, sem)

  drain(buf0, sem0)
  drain(buf1, sem1)


@jax.jit
def _launch(row, col, qid_i, qid_j, table_flat):
  mesh = plsc.VectorSubcoreMesh(core_axis_name="c", subcore_axis_name="s")
  cp = pltpu.CompilerParams()
  if "needs_layout_passes" in pltpu.CompilerParams.__dataclass_fields__:
    cp = dataclasses.replace(cp, needs_layout_passes=False)
  run = pl.kernel(
      _sc_body,
      compiler_params=cp,
      out_type=jax.ShapeDtypeStruct((H, T, T), jnp.float32),
      mesh=mesh,
      scratch_types=[
          pltpu.VMEM((REL * WPAD + 16,), jnp.int32),
          pltpu.VMEM((T,), jnp.int32),
          pltpu.VMEM((T,), jnp.int32),
          pltpu.VMEM((T,), jnp.int32),
          pltpu.VMEM((T,), jnp.int32),
          pltpu.VMEM((H, T), jnp.float32),
          pltpu.VMEM((H, T), jnp.float32),
          pltpu.SemaphoreType.DMA,
          pltpu.SemaphoreType.DMA,
          pltpu.SemaphoreType.DMA,
      ],
  )
  return run(row, col, qid_i, qid_j, table_flat)


def kernel(demo_row, demo_col, demo_id, is_sep, bias_weight):
  row = demo_row.astype(jnp.int32)
  col = demo_col.astype(jnp.int32)
  did = demo_id.astype(jnp.int32)
  invalid = is_sep | (did < 0)
  qid_j = jnp.where(invalid, -1, did)
  qid_i = jnp.where(invalid, -2, did)
  wb = jax.lax.bitcast_convert_type(
      bias_weight.astype(jnp.bfloat16), jnp.uint16).astype(jnp.uint32)
  words = wb[:, 0::2] | (wb[:, 1::2] << 16)
  words = jnp.pad(words, ((0, 0), (0, WPAD - H // 2)))
  table_flat = jax.lax.bitcast_convert_type(
      jnp.concatenate(
          [words.reshape(REL * WPAD), jnp.zeros((16,), jnp.uint32)]),
      jnp.int32)
  return _launch(row, col, qid_i, qid_j, table_flat)


# speedup vs baseline: 1.3508x; 1.3508x over previous
import dataclasses

import jax
import jax.numpy as jnp
from jax import lax
from jax.experimental import pallas as pl
from jax.experimental.pallas import tpu as pltpu
from jax.experimental.pallas import tpu_sc as plsc

G = 30
H = 12
T = 2048
SPAN_C = 4 * G + 1
REL = (2 * G - 1) * SPAN_C

NUM_CORES = 2
NUM_SUBCORES = 16
NW = NUM_CORES * NUM_SUBCORES
ROWS_PER = T // NW
LANES = 16
CHUNKS = T // LANES
WPAD = 7


def _sc_body(row_hbm, col_hbm, qi_hbm, qj_hbm, tab_hbm, out_hbm,
             tab_v, row_v, col_v, qj_v, qi_v, buf0, buf1,
             sem_in, sem0, sem1):
  cid = lax.axis_index("c")
  sid = lax.axis_index("s")
  wid = sid * NUM_CORES + cid

  stage = [pltpu.make_async_copy(src, dst, sem_in)
           for src, dst in ((tab_hbm, tab_v), (row_hbm, row_v),
                            (col_hbm, col_v), (qj_hbm, qj_v), (qi_hbm, qi_v))]
  for cp_ in stage:
    cp_.start()
  for cp_ in stage:
    cp_.wait()

  def fill_and_send(r, buf, sem):
    i = r * NW + wid
    bidx = jnp.zeros((LANES,), jnp.int32) + i
    bri = plsc.load_gather(row_v, [bidx])
    bci = plsc.load_gather(col_v, [bidx])
    bqi = plsc.load_gather(qi_v, [bidx])

    @plsc.parallel_loop(0, CHUNKS, unroll=4)
    def _(c):
      j0 = c * LANES
      rj = row_v[pl.ds(j0, LANES)]
      cj = col_v[pl.ds(j0, LANES)]
      qj = qj_v[pl.ds(j0, LANES)]
      dr = jnp.clip(bri - rj, -(G - 1), G - 1) + (G - 1)
      dc = jnp.clip(bci - cj, -2 * G, 2 * G) + 2 * G
      base = jnp.where(bqi == qj, (dr * SPAN_C + dc) * WPAD, REL * WPAD)
      for p in range(H // 2):
        w = plsc.load_gather(tab_v, [base + p])
        buf[2 * p, pl.ds(c * LANES, LANES)] = plsc.bitcast(
            w << 16, jnp.float32)
        buf[2 * p + 1, pl.ds(c * LANES, LANES)] = plsc.bitcast(
            w & jnp.int32(-65536), jnp.float32)

    pltpu.make_async_copy(buf, out_hbm.at[:, i, :], sem).start()

  def drain(buf, sem):
    pltpu.make_async_copy(buf, out_hbm.at[:, 0, :], sem).wait()

  @pl.loop(0, ROWS_PER, step=2)
  def _(r):
    for d, buf, sem in ((0, buf0, sem0), (1, buf1, sem1)):
      @pl.when(r > 0)
      def _():
        drain(buf, sem)
      fill_and_send(r + d, buf, sem)

  drain(buf0, sem0)
  drain(buf1, sem1)


@jax.jit
def _launch(row, col, qid_i, qid_j, table_flat):
  mesh = plsc.VectorSubcoreMesh(core_axis_name="c", subcore_axis_name="s")
  cp = pltpu.CompilerParams()
  if "needs_layout_passes" in pltpu.CompilerParams.__dataclass_fields__:
    cp = dataclasses.replace(cp, needs_layout_passes=False)
  run = pl.kernel(
      _sc_body,
      compiler_params=cp,
      out_type=jax.ShapeDtypeStruct((H, T, T), jnp.float32),
      mesh=mesh,
      scratch_types=[
          pltpu.VMEM((REL * WPAD + 16,), jnp.int32),
          pltpu.VMEM((T,), jnp.int32),
          pltpu.VMEM((T,), jnp.int32),
          pltpu.VMEM((T,), jnp.int32),
          pltpu.VMEM((T,), jnp.int32),
          pltpu.VMEM((H, T), jnp.float32),
          pltpu.VMEM((H, T), jnp.float32),
          pltpu.SemaphoreType.DMA,
          pltpu.SemaphoreType.DMA,
          pltpu.SemaphoreType.DMA,
      ],
  )
  return run(row, col, qid_i, qid_j, table_flat)


def kernel(demo_row, demo_col, demo_id, is_sep, bias_weight):
  row = demo_row.astype(jnp.int32)
  col = demo_col.astype(jnp.int32)
  did = demo_id.astype(jnp.int32)
  invalid = is_sep | (did < 0)
  qid_j = jnp.where(invalid, -1, did)
  qid_i = jnp.where(invalid, -2, did)
  wb = jax.lax.bitcast_convert_type(
      bias_weight.astype(jnp.bfloat16), jnp.uint16).astype(jnp.uint32)
  words = wb[:, 0::2] | (wb[:, 1::2] << 16)
  words = jnp.pad(words, ((0, 0), (0, WPAD - H // 2)))
  table_flat = jax.lax.bitcast_convert_type(
      jnp.concatenate(
          [words.reshape(REL * WPAD), jnp.zeros((16,), jnp.uint32)]),
      jnp.int32)
  return _launch(row, col, qid_i, qid_j, table_flat)


# speedup vs baseline: 1.3796x; 1.0213x over previous
import dataclasses

import jax
import jax.numpy as jnp
from jax import lax
from jax.experimental import pallas as pl
from jax.experimental.pallas import tpu as pltpu
from jax.experimental.pallas import tpu_sc as plsc

G = 30
H = 12
T = 2048
SPAN_C = 4 * G + 1
REL = (2 * G - 1) * SPAN_C

NUM_CORES = 2
NUM_SUBCORES = 16
NW = NUM_CORES * NUM_SUBCORES
ROWS_PER = T // NW
LANES = 16
CHUNKS = T // LANES
WPAD = 7


def _sc_body(row_hbm, col_hbm, qi_hbm, qj_hbm, tab_hbm, out_hbm,
             tab_v, tab_sh, row_v, col_v, qj_v, qi_v, buf0, buf1,
             sem_in, sem0, sem1):
  cid = lax.axis_index("c")
  sid = lax.axis_index("s")
  wid = sid * NUM_CORES + cid

  @pl.when(sid == 0)
  def _():
    pltpu.async_copy(tab_hbm, tab_sh, sem_in).wait()
  stage = [pltpu.make_async_copy(src, dst, sem_in)
           for src, dst in ((row_hbm, row_v), (col_hbm, col_v),
                            (qj_hbm, qj_v), (qi_hbm, qi_v))]
  for cp_ in stage:
    cp_.start()
  for cp_ in stage:
    cp_.wait()
  plsc.subcore_barrier()
  pltpu.sync_copy(tab_sh, tab_v)

  def fill_and_send(r, buf, sem):
    i = r * NW + wid
    bidx = jnp.zeros((LANES,), jnp.int32) + i
    bri = plsc.load_gather(row_v, [bidx])
    bci = plsc.load_gather(col_v, [bidx])
    bqi = plsc.load_gather(qi_v, [bidx])

    @plsc.parallel_loop(0, CHUNKS, unroll=4)
    def _(c):
      j0 = c * LANES
      rj = row_v[pl.ds(j0, LANES)]
      cj = col_v[pl.ds(j0, LANES)]
      qj = qj_v[pl.ds(j0, LANES)]
      dr = jnp.clip(bri - rj, -(G - 1), G - 1) + (G - 1)
      dc = jnp.clip(bci - cj, -2 * G, 2 * G) + 2 * G
      base = jnp.where(bqi == qj, (dr * SPAN_C + dc) * WPAD, REL * WPAD)
      for p in range(H // 2):
        w = plsc.load_gather(tab_v, [base + p])
        buf[2 * p, pl.ds(c * LANES, LANES)] = plsc.bitcast(
            w << 16, jnp.float32)
        buf[2 * p + 1, pl.ds(c * LANES, LANES)] = plsc.bitcast(
            w & jnp.int32(-65536), jnp.float32)

    pltpu.make_async_copy(buf, out_hbm.at[:, i, :], sem).start()

  def drain(buf, sem):
    pltpu.make_async_copy(buf, out_hbm.at[:, 0, :], sem).wait()

  @pl.loop(0, ROWS_PER, step=2)
  def _(r):
    for d, buf, sem in ((0, buf0, sem0), (1, buf1, sem1)):
      @pl.when(r > 0)
      def _():
        drain(buf, sem)
      fill_and_send(r + d, buf, sem)

  drain(buf0, sem0)
  drain(buf1, sem1)


@jax.jit
def _launch(row, col, qid_i, qid_j, table_flat):
  mesh = plsc.VectorSubcoreMesh(core_axis_name="c", subcore_axis_name="s")
  cp = pltpu.CompilerParams()
  if "needs_layout_passes" in pltpu.CompilerParams.__dataclass_fields__:
    cp = dataclasses.replace(cp, needs_layout_passes=False)
  run = pl.kernel(
      _sc_body,
      compiler_params=cp,
      out_type=jax.ShapeDtypeStruct((H, T, T), jnp.float32),
      mesh=mesh,
      scratch_types=[
          pltpu.VMEM((REL * WPAD + 16,), jnp.int32),
          pltpu.VMEM_SHARED((REL * WPAD + 16,), jnp.int32),
          pltpu.VMEM((T,), jnp.int32),
          pltpu.VMEM((T,), jnp.int32),
          pltpu.VMEM((T,), jnp.int32),
          pltpu.VMEM((T,), jnp.int32),
          pltpu.VMEM((H, T), jnp.float32),
          pltpu.VMEM((H, T), jnp.float32),
          pltpu.SemaphoreType.DMA,
          pltpu.SemaphoreType.DMA,
          pltpu.SemaphoreType.DMA,
      ],
  )
  return run(row, col, qid_i, qid_j, table_flat)


def kernel(demo_row, demo_col, demo_id, is_sep, bias_weight):
  row = demo_row.astype(jnp.int32)
  col = demo_col.astype(jnp.int32)
  did = demo_id.astype(jnp.int32)
  invalid = is_sep | (did < 0)
  qid_j = jnp.where(invalid, -1, did)
  qid_i = jnp.where(invalid, -2, did)
  wb = jax.lax.bitcast_convert_type(
      bias_weight.astype(jnp.bfloat16), jnp.uint16).astype(jnp.uint32)
  words = wb[:, 0::2] | (wb[:, 1::2] << 16)
  words = jnp.pad(words, ((0, 0), (0, WPAD - H // 2)))
  table_flat = jax.lax.bitcast_convert_type(
      jnp.concatenate(
          [words.reshape(REL * WPAD), jnp.zeros((16,), jnp.uint32)]),
      jnp.int32)
  return _launch(row, col, qid_i, qid_j, table_flat)
